# Initial kernel scaffold; baseline (speedup 1.0000x reference)
import functools

import jax
import jax.numpy as jnp
from jax import lax
from jax.experimental import pallas as pl
from jax.experimental.pallas import tpu as pltpu
from jax.experimental.pallas import tpu_sc as plsc

N = 10000
E = 320000
D_IN = 256
D_H = 128
MSG_STEPS = 3
EPS = 1e-5

NC = 2
NS = 16
NW = NC * NS
EPW = E // NW
K = 125
CH = EPW // K
RPT = N // NS
ZR = 125

_mesh = plsc.VectorSubcoreMesh(core_axis_name="c", subcore_axis_name="s")


@functools.partial(
    pl.kernel,
    out_type=jax.ShapeDtypeStruct((NC, NS, RPT, D_H), jnp.float32),
    mesh=_mesh,
    scratch_types=[
        pltpu.VMEM((CH, K), jnp.int32),
        pltpu.VMEM((CH, K), jnp.int32),
        pltpu.VMEM((K, D_H), jnp.float32),
        pltpu.VMEM((K, D_H), jnp.float32),
        pltpu.VMEM_SHARED((N, D_H), jnp.float32),
        pltpu.SemaphoreType.DMA,
        pltpu.SemaphoreType.DMA,
    ],
)
def _sc_scatter(g_hbm, src_hbm, dst_hbm, out_hbm,
                src_v, dst_v, rows0, rows1, acc, sem0, sem1):
    c = lax.axis_index("c")
    s = lax.axis_index("s")
    wid = c * NS + s

    pltpu.sync_copy(src_hbm.at[wid], src_v)
    pltpu.sync_copy(dst_hbm.at[wid], dst_v)

    zero = jnp.zeros((16,), jnp.float32)

    def _zfill(i, _):
        for j in range(D_H // 16):
            rows0[i, pl.ds(j * 16, 16)] = zero
        return 0

    lax.fori_loop(0, K, _zfill, 0)
    for t in range(RPT // ZR):
        pltpu.sync_copy(rows0, acc.at[pl.ds(s * RPT + t * ZR, ZR)])
    plsc.subcore_barrier()

    pltpu.async_copy(g_hbm.at[src_v.at[0]], rows0, sem0)

    def _body(jj, _):
        j0 = jj * 2
        j1 = j0 + 1
        pltpu.async_copy(g_hbm.at[src_v.at[j1]], rows1, sem1)
        pltpu.make_async_copy(g_hbm.at[src_v.at[j0]], rows0, sem0).wait()
        pltpu.sync_copy(rows0, acc.at[dst_v.at[j0]], add=True)

        @pl.when(j0 + 2 < CH)
        def _():
            pltpu.async_copy(g_hbm.at[src_v.at[j0 + 2]], rows0, sem0)

        pltpu.make_async_copy(g_hbm.at[src_v.at[j1]], rows1, sem1).wait()
        pltpu.sync_copy(rows1, acc.at[dst_v.at[j1]], add=True)
        return 0

    lax.fori_loop(0, CH // 2, _body, 0)
    plsc.subcore_barrier()

    pltpu.sync_copy(acc.at[pl.ds(s * RPT, RPT)], out_hbm.at[c, s])


@functools.partial(
    pl.kernel,
    out_type=jax.ShapeDtypeStruct((NC, N), jnp.float32),
    mesh=_mesh,
    scratch_types=[
        pltpu.VMEM((CH, K), jnp.int32),
        pltpu.VMEM((640,), jnp.float32),
        pltpu.VMEM((128,), jnp.float32),
        pltpu.VMEM_SHARED((N,), jnp.float32),
    ],
)
def _sc_degree(dst_hbm, out_hbm, dst_v, zeros_v, ones_v, acc):
    c = lax.axis_index("c")
    s = lax.axis_index("s")
    wid = c * NS + s

    pltpu.sync_copy(dst_hbm.at[wid], dst_v)

    zero = jnp.zeros((16,), jnp.float32)
    one = jnp.ones((16,), jnp.float32)
    for j in range(640 // 16):
        zeros_v[pl.ds(j * 16, 16)] = zero
    for j in range(128 // 16):
        ones_v[pl.ds(j * 16, 16)] = one

    @pl.when(s == 0)
    def _():
        for t in range(15):
            pltpu.sync_copy(zeros_v, acc.at[pl.ds(t * 640, 640)])
        pltpu.sync_copy(zeros_v.at[pl.ds(0, 400)], acc.at[pl.ds(9600, 400)])

    plsc.subcore_barrier()

    def _body(j, _):
        pltpu.sync_copy(ones_v.at[pl.ds(0, K)], acc.at[dst_v.at[j]], add=True)
        return 0

    lax.fori_loop(0, CH, _body, 0)
    plsc.subcore_barrier()

    @pl.when(s == 0)
    def _():
        pltpu.sync_copy(acc, out_hbm.at[c])


R = 1000
GRID = N // R


def _dinv(d0, d1):
    return lax.rsqrt(d0 + d1 + 1.0)


def _enc_body(x_ref, w_ref, b_ref, cw_ref, d0_ref, d1_ref, xo_ref, go_ref):
    x = jnp.dot(x_ref[...], w_ref[...],
                preferred_element_type=jnp.float32) + b_ref[...]
    xo_ref[...] = x
    dinv = _dinv(d0_ref[...], d1_ref[...])
    go_ref[...] = jnp.dot(x, cw_ref[...],
                          preferred_element_type=jnp.float32) * dinv


def _layer_norm(m, lg, lb):
    mu = jnp.mean(m, axis=-1, keepdims=True)
    var = jnp.mean((m - mu) ** 2, axis=-1, keepdims=True)
    return (m - mu) * lax.rsqrt(var + EPS) * lg + lb


def _step_body(x_ref, g_ref, p0_ref, p1_ref, d0_ref, d1_ref,
               cb_ref, lg_ref, lb_ref, cw_ref, xo_ref, go_ref):
    dinv = _dinv(d0_ref[...], d1_ref[...])
    m = x_ref[...] + dinv * (p0_ref[...] + p1_ref[...] + g_ref[...]) + cb_ref[...]
    xn = _layer_norm(m, lg_ref[...], lb_ref[...])
    xo_ref[...] = xn
    go_ref[...] = jnp.dot(xn, cw_ref[...],
                          preferred_element_type=jnp.float32) * dinv


def _final_body(x_ref, g_ref, p0_ref, p1_ref, d0_ref, d1_ref,
                cb_ref, lg_ref, lb_ref, dw_ref, db_ref, o_ref):
    dinv = _dinv(d0_ref[...], d1_ref[...])
    m = x_ref[...] + dinv * (p0_ref[...] + p1_ref[...] + g_ref[...]) + cb_ref[...]
    xn = _layer_norm(m, lg_ref[...], lb_ref[...])
    o_ref[...] = jnp.dot(xn, dw_ref[...],
                         preferred_element_type=jnp.float32) + db_ref[...]


def _row_spec(d):
    return pl.BlockSpec((R, d), lambda i: (i, 0))


def _full_spec(r, d):
    return pl.BlockSpec((r, d), lambda i: (0, 0))


def kernel(X, edge_index, enc_W, enc_b, conv_W, conv_b, ln_g, ln_b, dec_W, dec_b):
    src3 = edge_index[0].reshape(NW, CH, K)
    dst3 = edge_index[1].reshape(NW, CH, K)

    degp = _sc_degree(dst3)
    d0 = degp[0].reshape(N, 1)
    d1 = degp[1].reshape(N, 1)

    enc_b2 = enc_b.reshape(1, D_H)
    conv_b2 = conv_b.reshape(1, D_H)
    ln_g2 = ln_g.reshape(1, D_H)
    ln_b2 = ln_b.reshape(1, D_H)
    dec_b2 = dec_b.reshape(1, D_IN)

    x, g = pl.pallas_call(
        _enc_body,
        grid=(GRID,),
        in_specs=[
            _row_spec(D_IN),
            _full_spec(D_IN, D_H),
            _full_spec(1, D_H),
            _full_spec(D_H, D_H),
            _row_spec(1),
            _row_spec(1),
        ],
        out_specs=[_row_spec(D_H), _row_spec(D_H)],
        out_shape=[
            jax.ShapeDtypeStruct((N, D_H), jnp.float32),
            jax.ShapeDtypeStruct((N, D_H), jnp.float32),
        ],
    )(X, enc_W, enc_b2, conv_W, d0, d1)

    step_call = pl.pallas_call(
        _step_body,
        grid=(GRID,),
        in_specs=[
            _row_spec(D_H), _row_spec(D_H), _row_spec(D_H), _row_spec(D_H),
            _row_spec(1), _row_spec(1),
            _full_spec(1, D_H), _full_spec(1, D_H), _full_spec(1, D_H),
            _full_spec(D_H, D_H),
        ],
        out_specs=[_row_spec(D_H), _row_spec(D_H)],
        out_shape=[
            jax.ShapeDtypeStruct((N, D_H), jnp.float32),
            jax.ShapeDtypeStruct((N, D_H), jnp.float32),
        ],
    )

    final_call = pl.pallas_call(
        _final_body,
        grid=(GRID,),
        in_specs=[
            _row_spec(D_H), _row_spec(D_H), _row_spec(D_H), _row_spec(D_H),
            _row_spec(1), _row_spec(1),
            _full_spec(1, D_H), _full_spec(1, D_H), _full_spec(1, D_H),
            _full_spec(D_H, D_IN), _full_spec(1, D_IN),
        ],
        out_specs=[_row_spec(D_IN)],
        out_shape=jax.ShapeDtypeStruct((N, D_IN), jnp.float32),
    )

    for step in range(MSG_STEPS):
        part = _sc_scatter(g, src3, dst3)
        p = part.reshape(NC, N, D_H)
        if step < MSG_STEPS - 1:
            x, g = step_call(x, g, p[0], p[1], d0, d1,
                             conv_b2, ln_g2, ln_b2, conv_W)
        else:
            out = final_call(x, g, p[0], p[1], d0, d1,
                             conv_b2, ln_g2, ln_b2, dec_W, dec_b2)
    return out


# speedup vs baseline: 16.5509x; 16.5509x over previous
import functools

import jax
import jax.numpy as jnp
from jax import lax
from jax.experimental import pallas as pl
from jax.experimental.pallas import tpu as pltpu
from jax.experimental.pallas import tpu_sc as plsc

N = 10000
E = 320000
D_IN = 256
D_H = 128
MSG_STEPS = 3
EPS = 1e-5

NC = 2
NS = 16
NW = NC * NS
EPW = E // NW
K = 125
CH = EPW // K
RPT = N // NS
ZR = 125

_mesh = plsc.VectorSubcoreMesh(core_axis_name="c", subcore_axis_name="s")


@functools.partial(
    pl.kernel,
    out_type=jax.ShapeDtypeStruct((NC, NS, RPT, D_H), jnp.float32),
    mesh=_mesh,
    scratch_types=[
        pltpu.VMEM((CH, K), jnp.int32),
        pltpu.VMEM((CH, K), jnp.int32),
        pltpu.VMEM((K, D_H), jnp.float32),
        pltpu.VMEM((K, D_H), jnp.float32),
        pltpu.VMEM_SHARED((N, D_H), jnp.float32),
        pltpu.SemaphoreType.DMA,
        pltpu.SemaphoreType.DMA,
    ],
)
def _sc_scatter(g_hbm, src_hbm, dst_hbm, out_hbm,
                src_v, dst_v, rows0, rows1, acc, sem0, sem1):
    c = lax.axis_index("c")
    s = lax.axis_index("s")
    wid = c * NS + s

    pltpu.sync_copy(src_hbm.at[wid], src_v)
    pltpu.sync_copy(dst_hbm.at[wid], dst_v)

    zero = jnp.zeros((16,), jnp.float32)

    def _zfill(i, _):
        for j in range(D_H // 16):
            rows0[i, pl.ds(j * 16, 16)] = zero
        return 0

    lax.fori_loop(0, K, _zfill, 0)
    for t in range(RPT // ZR):
        pltpu.sync_copy(rows0, acc.at[pl.ds(s * RPT + t * ZR, ZR)])
    plsc.subcore_barrier()

    def _body(j, _):
        pltpu.async_copy(g_hbm.at[src_v.at[j]], rows0, sem0).wait()
        pltpu.sync_copy(rows0, acc.at[dst_v.at[j]], add=True)
        return 0

    lax.fori_loop(0, CH, _body, 0)
    plsc.subcore_barrier()

    pltpu.sync_copy(acc.at[pl.ds(s * RPT, RPT)], out_hbm.at[c, s])


@functools.partial(
    pl.kernel,
    out_type=jax.ShapeDtypeStruct((NC, N), jnp.float32),
    mesh=_mesh,
    scratch_types=[
        pltpu.VMEM((CH, K), jnp.int32),
        pltpu.VMEM((640,), jnp.float32),
        pltpu.VMEM((128,), jnp.float32),
        pltpu.VMEM_SHARED((N,), jnp.float32),
    ],
)
def _sc_degree(dst_hbm, out_hbm, dst_v, zeros_v, ones_v, acc):
    c = lax.axis_index("c")
    s = lax.axis_index("s")
    wid = c * NS + s

    pltpu.sync_copy(dst_hbm.at[wid], dst_v)

    zero = jnp.zeros((16,), jnp.float32)
    one = jnp.ones((16,), jnp.float32)
    for j in range(640 // 16):
        zeros_v[pl.ds(j * 16, 16)] = zero
    for j in range(128 // 16):
        ones_v[pl.ds(j * 16, 16)] = one

    @pl.when(s == 0)
    def _():
        for t in range(15):
            pltpu.sync_copy(zeros_v, acc.at[pl.ds(t * 640, 640)])
        pltpu.sync_copy(zeros_v.at[pl.ds(0, 400)], acc.at[pl.ds(9600, 400)])

    plsc.subcore_barrier()

    def _body(j, _):
        pltpu.sync_copy(ones_v.at[pl.ds(0, K)], acc.at[dst_v.at[j]], add=True)
        return 0

    lax.fori_loop(0, CH, _body, 0)
    plsc.subcore_barrier()

    @pl.when(s == 0)
    def _():
        pltpu.sync_copy(acc, out_hbm.at[c])


R = 1000
GRID = N // R


def _dinv(d0, d1):
    return lax.rsqrt(d0 + d1 + 1.0)


def _enc_body(x_ref, w_ref, b_ref, cw_ref, d0_ref, d1_ref, xo_ref, go_ref):
    x = jnp.dot(x_ref[...], w_ref[...],
                preferred_element_type=jnp.float32) + b_ref[...]
    xo_ref[...] = x
    dinv = _dinv(d0_ref[...], d1_ref[...])
    go_ref[...] = jnp.dot(x, cw_ref[...],
                          preferred_element_type=jnp.float32) * dinv


def _layer_norm(m, lg, lb):
    mu = jnp.mean(m, axis=-1, keepdims=True)
    var = jnp.mean((m - mu) ** 2, axis=-1, keepdims=True)
    return (m - mu) * lax.rsqrt(var + EPS) * lg + lb


def _step_body(x_ref, g_ref, p0_ref, p1_ref, d0_ref, d1_ref,
               cb_ref, lg_ref, lb_ref, cw_ref, xo_ref, go_ref):
    dinv = _dinv(d0_ref[...], d1_ref[...])
    m = x_ref[...] + dinv * (p0_ref[...] + p1_ref[...] + g_ref[...]) + cb_ref[...]
    xn = _layer_norm(m, lg_ref[...], lb_ref[...])
    xo_ref[...] = xn
    go_ref[...] = jnp.dot(xn, cw_ref[...],
                          preferred_element_type=jnp.float32) * dinv


def _final_body(x_ref, g_ref, p0_ref, p1_ref, d0_ref, d1_ref,
                cb_ref, lg_ref, lb_ref, dw_ref, db_ref, o_ref):
    dinv = _dinv(d0_ref[...], d1_ref[...])
    m = x_ref[...] + dinv * (p0_ref[...] + p1_ref[...] + g_ref[...]) + cb_ref[...]
    xn = _layer_norm(m, lg_ref[...], lb_ref[...])
    o_ref[...] = jnp.dot(xn, dw_ref[...],
                         preferred_element_type=jnp.float32) + db_ref[...]


def _row_spec(d):
    return pl.BlockSpec((R, d), lambda i: (i, 0))


def _full_spec(r, d):
    return pl.BlockSpec((r, d), lambda i: (0, 0))


def kernel(X, edge_index, enc_W, enc_b, conv_W, conv_b, ln_g, ln_b, dec_W, dec_b):
    src3 = edge_index[0].reshape(NW, CH, K)
    dst3 = edge_index[1].reshape(NW, CH, K)

    degp = _sc_degree(dst3)
    d0 = degp[0].reshape(N, 1)
    d1 = degp[1].reshape(N, 1)

    enc_b2 = enc_b.reshape(1, D_H)
    conv_b2 = conv_b.reshape(1, D_H)
    ln_g2 = ln_g.reshape(1, D_H)
    ln_b2 = ln_b.reshape(1, D_H)
    dec_b2 = dec_b.reshape(1, D_IN)

    x, g = pl.pallas_call(
        _enc_body,
        grid=(GRID,),
        in_specs=[
            _row_spec(D_IN),
            _full_spec(D_IN, D_H),
            _full_spec(1, D_H),
            _full_spec(D_H, D_H),
            _row_spec(1),
            _row_spec(1),
        ],
        out_specs=[_row_spec(D_H), _row_spec(D_H)],
        out_shape=[
            jax.ShapeDtypeStruct((N, D_H), jnp.float32),
            jax.ShapeDtypeStruct((N, D_H), jnp.float32),
        ],
    )(X, enc_W, enc_b2, conv_W, d0, d1)

    step_call = pl.pallas_call(
        _step_body,
        grid=(GRID,),
        in_specs=[
            _row_spec(D_H), _row_spec(D_H), _row_spec(D_H), _row_spec(D_H),
            _row_spec(1), _row_spec(1),
            _full_spec(1, D_H), _full_spec(1, D_H), _full_spec(1, D_H),
            _full_spec(D_H, D_H),
        ],
        out_specs=[_row_spec(D_H), _row_spec(D_H)],
        out_shape=[
            jax.ShapeDtypeStruct((N, D_H), jnp.float32),
            jax.ShapeDtypeStruct((N, D_H), jnp.float32),
        ],
    )

    final_call = pl.pallas_call(
        _final_body,
        grid=(GRID,),
        in_specs=[
            _row_spec(D_H), _row_spec(D_H), _row_spec(D_H), _row_spec(D_H),
            _row_spec(1), _row_spec(1),
            _full_spec(1, D_H), _full_spec(1, D_H), _full_spec(1, D_H),
            _full_spec(D_H, D_IN), _full_spec(1, D_IN),
        ],
        out_specs=_row_spec(D_IN),
        out_shape=jax.ShapeDtypeStruct((N, D_IN), jnp.float32),
    )

    for step in range(MSG_STEPS):
        part = _sc_scatter(g, src3, dst3)
        p = part.reshape(NC, N, D_H)
        if step < MSG_STEPS - 1:
            x, g = step_call(x, g, p[0], p[1], d0, d1,
                             conv_b2, ln_g2, ln_b2, conv_W)
        else:
            out = final_call(x, g, p[0], p[1], d0, d1,
                             conv_b2, ln_g2, ln_b2, dec_W, dec_b2)
    return out


# speedup vs baseline: 20.7583x; 1.2542x over previous
import functools

import jax
import jax.numpy as jnp
from jax import lax
from jax.experimental import pallas as pl
from jax.experimental.pallas import tpu as pltpu
from jax.experimental.pallas import tpu_sc as plsc

N = 10000
E = 320000
D_IN = 256
D_H = 128
MSG_STEPS = 3
EPS = 1e-5

NC = 2
NS = 16
NW = NC * NS
EPW = E // NW
K = 125
CH = EPW // K
RPT = N // NS
ZR = 125

_mesh = plsc.VectorSubcoreMesh(core_axis_name="c", subcore_axis_name="s")


@functools.partial(
    pl.kernel,
    out_type=jax.ShapeDtypeStruct((NC, NS, RPT, D_H), jnp.float32),
    mesh=_mesh,
    scratch_types=[
        pltpu.VMEM((CH, K), jnp.int32),
        pltpu.VMEM((K,), jnp.int32),
        pltpu.VMEM((K,), jnp.int32),
        pltpu.VMEM((K, D_H), jnp.float32),
        pltpu.VMEM((K, D_H), jnp.float32),
        pltpu.VMEM_SHARED((N, D_H), jnp.float32),
        pltpu.SemaphoreType.DMA,
        pltpu.SemaphoreType.DMA,
        pltpu.SemaphoreType.DMA,
        pltpu.SemaphoreType.DMA,
    ],
)
def _sc_scatter(g_hbm, src_hbm, dst_hbm, out_hbm,
                dst_v, si0, si1, rows0, rows1, acc,
                gsem0, gsem1, isem0, isem1):
    c = lax.axis_index("c")
    s = lax.axis_index("s")
    wid = c * NS + s

    pltpu.sync_copy(dst_hbm.at[wid], dst_v)

    zero = jnp.zeros((16,), jnp.float32)

    def _zfill(i, _):
        for j in range(D_H // 16):
            rows0[i, pl.ds(j * 16, 16)] = zero
        return 0

    lax.fori_loop(0, ZR, _zfill, 0)
    for t in range(RPT // ZR):
        pltpu.sync_copy(rows0, acc.at[pl.ds(s * RPT + t * ZR, ZR)])
    plsc.subcore_barrier()

    pltpu.sync_copy(src_hbm.at[wid, 0], si0)
    pltpu.async_copy(g_hbm.at[si0], rows0, gsem0)
    pltpu.async_copy(src_hbm.at[wid, 1], si1, isem1)

    def _body(jj, _):
        j0 = jj * 2
        j1 = j0 + 1
        pltpu.make_async_copy(g_hbm.at[si0], rows0, gsem0).wait()
        pltpu.make_async_copy(src_hbm.at[wid, j1], si1, isem1).wait()
        pltpu.async_copy(g_hbm.at[si1], rows1, gsem1)

        @pl.when(j0 + 2 < CH)
        def _():
            pltpu.async_copy(src_hbm.at[wid, j0 + 2], si0, isem0)

        pltpu.sync_copy(rows0, acc.at[dst_v.at[j0]], add=True)

        pltpu.make_async_copy(g_hbm.at[si1], rows1, gsem1).wait()

        @pl.when(j0 + 2 < CH)
        def _():
            pltpu.make_async_copy(src_hbm.at[wid, j0 + 2], si0, isem0).wait()
            pltpu.async_copy(g_hbm.at[si0], rows0, gsem0)
            pltpu.async_copy(src_hbm.at[wid, j0 + 3], si1, isem1)

        pltpu.sync_copy(rows1, acc.at[dst_v.at[j1]], add=True)
        return 0

    lax.fori_loop(0, CH // 2, _body, 0)
    plsc.subcore_barrier()

    pltpu.sync_copy(acc.at[pl.ds(s * RPT, RPT)], out_hbm.at[c, s])


@functools.partial(
    pl.kernel,
    out_type=jax.ShapeDtypeStruct((NC, N), jnp.float32),
    mesh=_mesh,
    scratch_types=[
        pltpu.VMEM((CH, K), jnp.int32),
        pltpu.VMEM((640,), jnp.float32),
        pltpu.VMEM((128,), jnp.float32),
        pltpu.VMEM_SHARED((N,), jnp.float32),
    ],
)
def _sc_degree(dst_hbm, out_hbm, dst_v, zeros_v, ones_v, acc):
    c = lax.axis_index("c")
    s = lax.axis_index("s")
    wid = c * NS + s

    pltpu.sync_copy(dst_hbm.at[wid], dst_v)

    zero = jnp.zeros((16,), jnp.float32)
    one = jnp.ones((16,), jnp.float32)
    for j in range(640 // 16):
        zeros_v[pl.ds(j * 16, 16)] = zero
    for j in range(128 // 16):
        ones_v[pl.ds(j * 16, 16)] = one

    @pl.when(s == 0)
    def _():
        for t in range(15):
            pltpu.sync_copy(zeros_v, acc.at[pl.ds(t * 640, 640)])
        pltpu.sync_copy(zeros_v.at[pl.ds(0, 400)], acc.at[pl.ds(9600, 400)])

    plsc.subcore_barrier()

    def _body(j, _):
        pltpu.sync_copy(ones_v.at[pl.ds(0, K)], acc.at[dst_v.at[j]], add=True)
        return 0

    lax.fori_loop(0, CH, _body, 0)
    plsc.subcore_barrier()

    @pl.when(s == 0)
    def _():
        pltpu.sync_copy(acc, out_hbm.at[c])


R = 1000
GRID = N // R


def _dinv(d0, d1):
    return lax.rsqrt(d0 + d1 + 1.0)


def _enc_body(x_ref, w_ref, b_ref, cw_ref, d0_ref, d1_ref, xo_ref, go_ref):
    x = jnp.dot(x_ref[...], w_ref[...],
                preferred_element_type=jnp.float32) + b_ref[...]
    xo_ref[...] = x
    dinv = _dinv(d0_ref[...], d1_ref[...])
    go_ref[...] = jnp.dot(x, cw_ref[...],
                          preferred_element_type=jnp.float32) * dinv


def _layer_norm(m, lg, lb):
    mu = jnp.mean(m, axis=-1, keepdims=True)
    var = jnp.mean((m - mu) ** 2, axis=-1, keepdims=True)
    return (m - mu) * lax.rsqrt(var + EPS) * lg + lb


def _step_body(x_ref, g_ref, p0_ref, p1_ref, d0_ref, d1_ref,
               cb_ref, lg_ref, lb_ref, cw_ref, xo_ref, go_ref):
    dinv = _dinv(d0_ref[...], d1_ref[...])
    m = x_ref[...] + dinv * (p0_ref[...] + p1_ref[...] + g_ref[...]) + cb_ref[...]
    xn = _layer_norm(m, lg_ref[...], lb_ref[...])
    xo_ref[...] = xn
    go_ref[...] = jnp.dot(xn, cw_ref[...],
                          preferred_element_type=jnp.float32) * dinv


def _final_body(x_ref, g_ref, p0_ref, p1_ref, d0_ref, d1_ref,
                cb_ref, lg_ref, lb_ref, dw_ref, db_ref, o_ref):
    dinv = _dinv(d0_ref[...], d1_ref[...])
    m = x_ref[...] + dinv * (p0_ref[...] + p1_ref[...] + g_ref[...]) + cb_ref[...]
    xn = _layer_norm(m, lg_ref[...], lb_ref[...])
    o_ref[...] = jnp.dot(xn, dw_ref[...],
                         preferred_element_type=jnp.float32) + db_ref[...]


def _row_spec(d):
    return pl.BlockSpec((R, d), lambda i: (i, 0))


def _full_spec(r, d):
    return pl.BlockSpec((r, d), lambda i: (0, 0))


def kernel(X, edge_index, enc_W, enc_b, conv_W, conv_b, ln_g, ln_b, dec_W, dec_b):
    src3 = edge_index[0].reshape(NW, CH, K)
    dst3 = edge_index[1].reshape(NW, CH, K)

    degp = _sc_degree(dst3)
    d0 = degp[0].reshape(N, 1)
    d1 = degp[1].reshape(N, 1)

    enc_b2 = enc_b.reshape(1, D_H)
    conv_b2 = conv_b.reshape(1, D_H)
    ln_g2 = ln_g.reshape(1, D_H)
    ln_b2 = ln_b.reshape(1, D_H)
    dec_b2 = dec_b.reshape(1, D_IN)

    x, g = pl.pallas_call(
        _enc_body,
        grid=(GRID,),
        in_specs=[
            _row_spec(D_IN),
            _full_spec(D_IN, D_H),
            _full_spec(1, D_H),
            _full_spec(D_H, D_H),
            _row_spec(1),
            _row_spec(1),
        ],
        out_specs=[_row_spec(D_H), _row_spec(D_H)],
        out_shape=[
            jax.ShapeDtypeStruct((N, D_H), jnp.float32),
            jax.ShapeDtypeStruct((N, D_H), jnp.float32),
        ],
    )(X, enc_W, enc_b2, conv_W, d0, d1)

    step_call = pl.pallas_call(
        _step_body,
        grid=(GRID,),
        in_specs=[
            _row_spec(D_H), _row_spec(D_H), _row_spec(D_H), _row_spec(D_H),
            _row_spec(1), _row_spec(1),
            _full_spec(1, D_H), _full_spec(1, D_H), _full_spec(1, D_H),
            _full_spec(D_H, D_H),
        ],
        out_specs=[_row_spec(D_H), _row_spec(D_H)],
        out_shape=[
            jax.ShapeDtypeStruct((N, D_H), jnp.float32),
            jax.ShapeDtypeStruct((N, D_H), jnp.float32),
        ],
    )

    final_call = pl.pallas_call(
        _final_body,
        grid=(GRID,),
        in_specs=[
            _row_spec(D_H), _row_spec(D_H), _row_spec(D_H), _row_spec(D_H),
            _row_spec(1), _row_spec(1),
            _full_spec(1, D_H), _full_spec(1, D_H), _full_spec(1, D_H),
            _full_spec(D_H, D_IN), _full_spec(1, D_IN),
        ],
        out_specs=_row_spec(D_IN),
        out_shape=jax.ShapeDtypeStruct((N, D_IN), jnp.float32),
    )

    for step in range(MSG_STEPS):
        part = _sc_scatter(g, src3, dst3)
        p = part.reshape(NC, N, D_H)
        if step < MSG_STEPS - 1:
            x, g = step_call(x, g, p[0], p[1], d0, d1,
                             conv_b2, ln_g2, ln_b2, conv_W)
        else:
            out = final_call(x, g, p[0], p[1], d0, d1,
                             conv_b2, ln_g2, ln_b2, dec_W, dec_b2)
    return out


# speedup vs baseline: 21.0565x; 1.0144x over previous
import functools

import jax
import jax.numpy as jnp
from jax import lax
from jax.experimental import pallas as pl
from jax.experimental.pallas import tpu as pltpu
from jax.experimental.pallas import tpu_sc as plsc

N = 10000
E = 320000
D_IN = 256
D_H = 128
MSG_STEPS = 3
EPS = 1e-5

NC = 2
NS = 16
NW = NC * NS
EPW = E // NW
K = 125
CH = EPW // K
RPT = N // NS
ZR = 125

_mesh = plsc.VectorSubcoreMesh(core_axis_name="c", subcore_axis_name="s")


@functools.partial(
    pl.kernel,
    out_type=jax.ShapeDtypeStruct((NC, NS, RPT, D_H), jnp.float32),
    mesh=_mesh,
    scratch_types=[
        pltpu.VMEM((CH, K), jnp.int32),
        pltpu.VMEM((K,), jnp.int32),
        pltpu.VMEM((K,), jnp.int32),
        pltpu.VMEM((K, D_H), jnp.float32),
        pltpu.VMEM((K, D_H), jnp.float32),
        pltpu.VMEM_SHARED((N, D_H), jnp.float32),
        pltpu.SemaphoreType.DMA,
        pltpu.SemaphoreType.DMA,
        pltpu.SemaphoreType.DMA,
        pltpu.SemaphoreType.DMA,
        pltpu.SemaphoreType.DMA,
    ],
)
def _sc_scatter(g_hbm, src_hbm, dst_hbm, out_hbm,
                dst_v, si0, si1, rows0, rows1, acc,
                gsem0, gsem1, isem0, isem1, zsem):
    c = lax.axis_index("c")
    s = lax.axis_index("s")
    wid = c * NS + s

    zero = jnp.zeros((16,), jnp.float32)

    def _zfill(i, _):
        for j in range(D_H // 16):
            rows1[i, pl.ds(j * 16, 16)] = zero
        return 0

    lax.fori_loop(0, ZR, _zfill, 0)
    for t in range(RPT // ZR):
        pltpu.async_copy(rows1, acc.at[pl.ds(s * RPT + t * ZR, ZR)], zsem)

    pltpu.sync_copy(dst_hbm.at[wid], dst_v)
    pltpu.sync_copy(src_hbm.at[wid, 0], si0)
    pltpu.async_copy(g_hbm.at[si0], rows0, gsem0)
    pltpu.async_copy(src_hbm.at[wid, 1], si1, isem1)

    for t in range(RPT // ZR):
        pltpu.make_async_copy(
            rows1, acc.at[pl.ds(s * RPT + t * ZR, ZR)], zsem).wait()
    plsc.subcore_barrier()

    def _body(jj, _):
        j0 = jj * 2
        j1 = j0 + 1
        pltpu.make_async_copy(g_hbm.at[si0], rows0, gsem0).wait()
        pltpu.make_async_copy(src_hbm.at[wid, j1], si1, isem1).wait()
        pltpu.async_copy(g_hbm.at[si1], rows1, gsem1)

        @pl.when(j0 + 2 < CH)
        def _():
            pltpu.async_copy(src_hbm.at[wid, j0 + 2], si0, isem0)

        pltpu.sync_copy(rows0, acc.at[dst_v.at[j0]], add=True)

        pltpu.make_async_copy(g_hbm.at[si1], rows1, gsem1).wait()

        @pl.when(j0 + 2 < CH)
        def _():
            pltpu.make_async_copy(src_hbm.at[wid, j0 + 2], si0, isem0).wait()
            pltpu.async_copy(g_hbm.at[si0], rows0, gsem0)
            pltpu.async_copy(src_hbm.at[wid, j0 + 3], si1, isem1)

        pltpu.sync_copy(rows1, acc.at[dst_v.at[j1]], add=True)
        return 0

    lax.fori_loop(0, CH // 2, _body, 0)
    plsc.subcore_barrier()

    pltpu.sync_copy(acc.at[pl.ds(s * RPT, RPT)], out_hbm.at[c, s])


@functools.partial(
    pl.kernel,
    out_type=jax.ShapeDtypeStruct((NC, N), jnp.float32),
    mesh=_mesh,
    scratch_types=[
        pltpu.VMEM((CH, K), jnp.int32),
        pltpu.VMEM((640,), jnp.float32),
        pltpu.VMEM((128,), jnp.float32),
        pltpu.VMEM_SHARED((N,), jnp.float32),
    ],
)
def _sc_degree(dst_hbm, out_hbm, dst_v, zeros_v, ones_v, acc):
    c = lax.axis_index("c")
    s = lax.axis_index("s")
    wid = c * NS + s

    pltpu.sync_copy(dst_hbm.at[wid], dst_v)

    zero = jnp.zeros((16,), jnp.float32)
    one = jnp.ones((16,), jnp.float32)
    for j in range(640 // 16):
        zeros_v[pl.ds(j * 16, 16)] = zero
    for j in range(128 // 16):
        ones_v[pl.ds(j * 16, 16)] = one

    @pl.when(s == 0)
    def _():
        for t in range(15):
            pltpu.sync_copy(zeros_v, acc.at[pl.ds(t * 640, 640)])
        pltpu.sync_copy(zeros_v.at[pl.ds(0, 400)], acc.at[pl.ds(9600, 400)])

    plsc.subcore_barrier()

    def _body(j, _):
        pltpu.sync_copy(ones_v.at[pl.ds(0, K)], acc.at[dst_v.at[j]], add=True)
        return 0

    lax.fori_loop(0, CH, _body, 0)
    plsc.subcore_barrier()

    @pl.when(s == 0)
    def _():
        pltpu.sync_copy(acc, out_hbm.at[c])


R = 1000
GRID = N // R


def _dinv(d0, d1):
    return lax.rsqrt(d0 + d1 + 1.0)


def _enc_body(x_ref, w_ref, b_ref, cw_ref, d0_ref, d1_ref, xo_ref, go_ref):
    x = jnp.dot(x_ref[...], w_ref[...],
                preferred_element_type=jnp.float32) + b_ref[...]
    xo_ref[...] = x
    dinv = _dinv(d0_ref[...], d1_ref[...])
    go_ref[...] = jnp.dot(x, cw_ref[...],
                          preferred_element_type=jnp.float32) * dinv


def _layer_norm(m, lg, lb):
    mu = jnp.mean(m, axis=-1, keepdims=True)
    var = jnp.mean((m - mu) ** 2, axis=-1, keepdims=True)
    return (m - mu) * lax.rsqrt(var + EPS) * lg + lb


def _step_body(x_ref, g_ref, p0_ref, p1_ref, d0_ref, d1_ref,
               cb_ref, lg_ref, lb_ref, cw_ref, xo_ref, go_ref):
    dinv = _dinv(d0_ref[...], d1_ref[...])
    m = x_ref[...] + dinv * (p0_ref[...] + p1_ref[...] + g_ref[...]) + cb_ref[...]
    xn = _layer_norm(m, lg_ref[...], lb_ref[...])
    xo_ref[...] = xn
    go_ref[...] = jnp.dot(xn, cw_ref[...],
                          preferred_element_type=jnp.float32) * dinv


def _final_body(x_ref, g_ref, p0_ref, p1_ref, d0_ref, d1_ref,
                cb_ref, lg_ref, lb_ref, dw_ref, db_ref, o_ref):
    dinv = _dinv(d0_ref[...], d1_ref[...])
    m = x_ref[...] + dinv * (p0_ref[...] + p1_ref[...] + g_ref[...]) + cb_ref[...]
    xn = _layer_norm(m, lg_ref[...], lb_ref[...])
    o_ref[...] = jnp.dot(xn, dw_ref[...],
                         preferred_element_type=jnp.float32) + db_ref[...]


def _row_spec(d):
    return pl.BlockSpec((R, d), lambda i: (i, 0))


def _full_spec(r, d):
    return pl.BlockSpec((r, d), lambda i: (0, 0))


def kernel(X, edge_index, enc_W, enc_b, conv_W, conv_b, ln_g, ln_b, dec_W, dec_b):
    src3 = edge_index[0].reshape(NW, CH, K)
    dst3 = edge_index[1].reshape(NW, CH, K)

    degp = _sc_degree(dst3)
    d0 = degp[0].reshape(N, 1)
    d1 = degp[1].reshape(N, 1)

    enc_b2 = enc_b.reshape(1, D_H)
    conv_b2 = conv_b.reshape(1, D_H)
    ln_g2 = ln_g.reshape(1, D_H)
    ln_b2 = ln_b.reshape(1, D_H)
    dec_b2 = dec_b.reshape(1, D_IN)

    x, g = pl.pallas_call(
        _enc_body,
        grid=(GRID,),
        in_specs=[
            _row_spec(D_IN),
            _full_spec(D_IN, D_H),
            _full_spec(1, D_H),
            _full_spec(D_H, D_H),
            _row_spec(1),
            _row_spec(1),
        ],
        out_specs=[_row_spec(D_H), _row_spec(D_H)],
        out_shape=[
            jax.ShapeDtypeStruct((N, D_H), jnp.float32),
            jax.ShapeDtypeStruct((N, D_H), jnp.float32),
        ],
    )(X, enc_W, enc_b2, conv_W, d0, d1)

    step_call = pl.pallas_call(
        _step_body,
        grid=(GRID,),
        in_specs=[
            _row_spec(D_H), _row_spec(D_H), _row_spec(D_H), _row_spec(D_H),
            _row_spec(1), _row_spec(1),
            _full_spec(1, D_H), _full_spec(1, D_H), _full_spec(1, D_H),
            _full_spec(D_H, D_H),
        ],
        out_specs=[_row_spec(D_H), _row_spec(D_H)],
        out_shape=[
            jax.ShapeDtypeStruct((N, D_H), jnp.float32),
            jax.ShapeDtypeStruct((N, D_H), jnp.float32),
        ],
    )

    final_call = pl.pallas_call(
        _final_body,
        grid=(GRID,),
        in_specs=[
            _row_spec(D_H), _row_spec(D_H), _row_spec(D_H), _row_spec(D_H),
            _row_spec(1), _row_spec(1),
            _full_spec(1, D_H), _full_spec(1, D_H), _full_spec(1, D_H),
            _full_spec(D_H, D_IN), _full_spec(1, D_IN),
        ],
        out_specs=_row_spec(D_IN),
        out_shape=jax.ShapeDtypeStruct((N, D_IN), jnp.float32),
    )

    for step in range(MSG_STEPS):
        part = _sc_scatter(g, src3, dst3)
        p = part.reshape(NC, N, D_H)
        if step < MSG_STEPS - 1:
            x, g = step_call(x, g, p[0], p[1], d0, d1,
                             conv_b2, ln_g2, ln_b2, conv_W)
        else:
            out = final_call(x, g, p[0], p[1], d0, d1,
                             conv_b2, ln_g2, ln_b2, dec_W, dec_b2)
    return out


# speedup vs baseline: 24.0484x; 1.1421x over previous
import functools

import jax
import jax.numpy as jnp
from jax import lax
from jax.experimental import pallas as pl
from jax.experimental.pallas import tpu as pltpu
from jax.experimental.pallas import tpu_sc as plsc

N = 10000
E = 320000
D_IN = 256
D_H = 128
MSG_STEPS = 3
EPS = 1e-5

NC = 2
NS = 16
NW = NC * NS
EPW = E // NW
K = 125
CH = EPW // K
RPT = N // NS
ZR = 125

_mesh = plsc.VectorSubcoreMesh(core_axis_name="c", subcore_axis_name="s")


@functools.partial(
    pl.kernel,
    out_type=jax.ShapeDtypeStruct((NC, NS, RPT, D_H), jnp.float32),
    mesh=_mesh,
    scratch_types=[
        pltpu.VMEM((CH, K), jnp.int32),
        pltpu.VMEM((K,), jnp.int32),
        pltpu.VMEM((K,), jnp.int32),
        pltpu.VMEM((K, D_H), jnp.float32),
        pltpu.VMEM((K, D_H), jnp.float32),
        pltpu.VMEM_SHARED((N, D_H), jnp.float32),
        pltpu.SemaphoreType.DMA,
        pltpu.SemaphoreType.DMA,
        pltpu.SemaphoreType.DMA,
        pltpu.SemaphoreType.DMA,
        pltpu.SemaphoreType.DMA,
    ],
)
def _sc_scatter(g_hbm, src_hbm, dst_hbm, out_hbm,
                dst_v, si0, si1, rows0, rows1, acc,
                gsem0, gsem1, isem0, isem1, zsem):
    c = lax.axis_index("c")
    s = lax.axis_index("s")
    wid = c * NS + s

    zero = jnp.zeros((16,), jnp.float32)

    def _zfill(i, _):
        for j in range(D_H // 16):
            rows1[i, pl.ds(j * 16, 16)] = zero
        return 0

    lax.fori_loop(0, ZR, _zfill, 0)
    for t in range(RPT // ZR):
        pltpu.async_copy(rows1, acc.at[pl.ds(s * RPT + t * ZR, ZR)], zsem)

    pltpu.sync_copy(dst_hbm.at[wid], dst_v)
    pltpu.sync_copy(src_hbm.at[wid, 0], si0)
    pltpu.sync_copy(src_hbm.at[wid, 1], si1)
    pltpu.async_copy(g_hbm.at[si0], rows0, gsem0)

    for t in range(RPT // ZR):
        pltpu.make_async_copy(
            rows1, acc.at[pl.ds(s * RPT + t * ZR, ZR)], zsem).wait()
    pltpu.async_copy(g_hbm.at[si1], rows1, gsem1)
    plsc.subcore_barrier()

    def _body(jj, _):
        j0 = jj * 2
        j1 = j0 + 1
        pltpu.make_async_copy(g_hbm.at[si0], rows0, gsem0).wait()

        @pl.when(j0 + 2 < CH)
        def _():
            pltpu.async_copy(src_hbm.at[wid, j0 + 2], si0, isem0)

        pltpu.sync_copy(rows0, acc.at[dst_v.at[j0]], add=True)

        @pl.when(j0 + 2 < CH)
        def _():
            pltpu.make_async_copy(src_hbm.at[wid, j0 + 2], si0, isem0).wait()
            pltpu.async_copy(g_hbm.at[si0], rows0, gsem0)

        pltpu.make_async_copy(g_hbm.at[si1], rows1, gsem1).wait()

        @pl.when(j1 + 2 < CH)
        def _():
            pltpu.async_copy(src_hbm.at[wid, j1 + 2], si1, isem1)

        pltpu.sync_copy(rows1, acc.at[dst_v.at[j1]], add=True)

        @pl.when(j1 + 2 < CH)
        def _():
            pltpu.make_async_copy(src_hbm.at[wid, j1 + 2], si1, isem1).wait()
            pltpu.async_copy(g_hbm.at[si1], rows1, gsem1)

        return 0

    lax.fori_loop(0, CH // 2, _body, 0)
    plsc.subcore_barrier()

    pltpu.sync_copy(acc.at[pl.ds(s * RPT, RPT)], out_hbm.at[c, s])


@functools.partial(
    pl.kernel,
    out_type=jax.ShapeDtypeStruct((NC, N), jnp.float32),
    mesh=_mesh,
    scratch_types=[
        pltpu.VMEM((CH, K), jnp.int32),
        pltpu.VMEM((640,), jnp.float32),
        pltpu.VMEM((128,), jnp.float32),
        pltpu.VMEM_SHARED((N,), jnp.float32),
    ],
)
def _sc_degree(dst_hbm, out_hbm, dst_v, zeros_v, ones_v, acc):
    c = lax.axis_index("c")
    s = lax.axis_index("s")
    wid = c * NS + s

    pltpu.sync_copy(dst_hbm.at[wid], dst_v)

    zero = jnp.zeros((16,), jnp.float32)
    one = jnp.ones((16,), jnp.float32)
    for j in range(640 // 16):
        zeros_v[pl.ds(j * 16, 16)] = zero
    for j in range(128 // 16):
        ones_v[pl.ds(j * 16, 16)] = one

    @pl.when(s == 0)
    def _():
        for t in range(15):
            pltpu.sync_copy(zeros_v, acc.at[pl.ds(t * 640, 640)])
        pltpu.sync_copy(zeros_v.at[pl.ds(0, 400)], acc.at[pl.ds(9600, 400)])

    plsc.subcore_barrier()

    def _body(j, _):
        pltpu.sync_copy(ones_v.at[pl.ds(0, K)], acc.at[dst_v.at[j]], add=True)
        return 0

    lax.fori_loop(0, CH, _body, 0)
    plsc.subcore_barrier()

    @pl.when(s == 0)
    def _():
        pltpu.sync_copy(acc, out_hbm.at[c])


R = 1000
GRID = N // R


def _dinv(d0, d1):
    return lax.rsqrt(d0 + d1 + 1.0)


def _enc_body(x_ref, w_ref, b_ref, cw_ref, d0_ref, d1_ref, xo_ref, go_ref):
    x = jnp.dot(x_ref[...], w_ref[...],
                preferred_element_type=jnp.float32) + b_ref[...]
    xo_ref[...] = x
    dinv = _dinv(d0_ref[...], d1_ref[...])
    go_ref[...] = jnp.dot(x, cw_ref[...],
                          preferred_element_type=jnp.float32) * dinv


def _layer_norm(m, lg, lb):
    mu = jnp.mean(m, axis=-1, keepdims=True)
    var = jnp.mean((m - mu) ** 2, axis=-1, keepdims=True)
    return (m - mu) * lax.rsqrt(var + EPS) * lg + lb


def _step_body(x_ref, g_ref, p0_ref, p1_ref, d0_ref, d1_ref,
               cb_ref, lg_ref, lb_ref, cw_ref, xo_ref, go_ref):
    dinv = _dinv(d0_ref[...], d1_ref[...])
    m = x_ref[...] + dinv * (p0_ref[...] + p1_ref[...] + g_ref[...]) + cb_ref[...]
    xn = _layer_norm(m, lg_ref[...], lb_ref[...])
    xo_ref[...] = xn
    go_ref[...] = jnp.dot(xn, cw_ref[...],
                          preferred_element_type=jnp.float32) * dinv


def _final_body(x_ref, g_ref, p0_ref, p1_ref, d0_ref, d1_ref,
                cb_ref, lg_ref, lb_ref, dw_ref, db_ref, o_ref):
    dinv = _dinv(d0_ref[...], d1_ref[...])
    m = x_ref[...] + dinv * (p0_ref[...] + p1_ref[...] + g_ref[...]) + cb_ref[...]
    xn = _layer_norm(m, lg_ref[...], lb_ref[...])
    o_ref[...] = jnp.dot(xn, dw_ref[...],
                         preferred_element_type=jnp.float32) + db_ref[...]


def _row_spec(d):
    return pl.BlockSpec((R, d), lambda i: (i, 0))


def _full_spec(r, d):
    return pl.BlockSpec((r, d), lambda i: (0, 0))


def kernel(X, edge_index, enc_W, enc_b, conv_W, conv_b, ln_g, ln_b, dec_W, dec_b):
    src3 = edge_index[0].reshape(NW, CH, K)
    dst3 = edge_index[1].reshape(NW, CH, K)

    degp = _sc_degree(dst3)
    d0 = degp[0].reshape(N, 1)
    d1 = degp[1].reshape(N, 1)

    enc_b2 = enc_b.reshape(1, D_H)
    conv_b2 = conv_b.reshape(1, D_H)
    ln_g2 = ln_g.reshape(1, D_H)
    ln_b2 = ln_b.reshape(1, D_H)
    dec_b2 = dec_b.reshape(1, D_IN)

    x, g = pl.pallas_call(
        _enc_body,
        grid=(GRID,),
        in_specs=[
            _row_spec(D_IN),
            _full_spec(D_IN, D_H),
            _full_spec(1, D_H),
            _full_spec(D_H, D_H),
            _row_spec(1),
            _row_spec(1),
        ],
        out_specs=[_row_spec(D_H), _row_spec(D_H)],
        out_shape=[
            jax.ShapeDtypeStruct((N, D_H), jnp.float32),
            jax.ShapeDtypeStruct((N, D_H), jnp.float32),
        ],
    )(X, enc_W, enc_b2, conv_W, d0, d1)

    step_call = pl.pallas_call(
        _step_body,
        grid=(GRID,),
        in_specs=[
            _row_spec(D_H), _row_spec(D_H), _row_spec(D_H), _row_spec(D_H),
            _row_spec(1), _row_spec(1),
            _full_spec(1, D_H), _full_spec(1, D_H), _full_spec(1, D_H),
            _full_spec(D_H, D_H),
        ],
        out_specs=[_row_spec(D_H), _row_spec(D_H)],
        out_shape=[
            jax.ShapeDtypeStruct((N, D_H), jnp.float32),
            jax.ShapeDtypeStruct((N, D_H), jnp.float32),
        ],
    )

    final_call = pl.pallas_call(
        _final_body,
        grid=(GRID,),
        in_specs=[
            _row_spec(D_H), _row_spec(D_H), _row_spec(D_H), _row_spec(D_H),
            _row_spec(1), _row_spec(1),
            _full_spec(1, D_H), _full_spec(1, D_H), _full_spec(1, D_H),
            _full_spec(D_H, D_IN), _full_spec(1, D_IN),
        ],
        out_specs=_row_spec(D_IN),
        out_shape=jax.ShapeDtypeStruct((N, D_IN), jnp.float32),
    )

    for step in range(MSG_STEPS):
        part = _sc_scatter(g, src3, dst3)
        p = part.reshape(NC, N, D_H)
        if step < MSG_STEPS - 1:
            x, g = step_call(x, g, p[0], p[1], d0, d1,
                             conv_b2, ln_g2, ln_b2, conv_W)
        else:
            out = final_call(x, g, p[0], p[1], d0, d1,
                             conv_b2, ln_g2, ln_b2, dec_W, dec_b2)
    return out


# speedup vs baseline: 24.4594x; 1.0171x over previous
import functools

import jax
import jax.numpy as jnp
from jax import lax
from jax.experimental import pallas as pl
from jax.experimental.pallas import tpu as pltpu
from jax.experimental.pallas import tpu_sc as plsc

N = 10000
E = 320000
D_IN = 256
D_H = 128
MSG_STEPS = 3
EPS = 1e-5

NC = 2
NS = 16
NW = NC * NS
EPW = E // NW
K = 125
CH = EPW // K
RPT = N // NS
ZR = 125

_mesh = plsc.VectorSubcoreMesh(core_axis_name="c", subcore_axis_name="s")


@functools.partial(
    pl.kernel,
    out_type=jax.ShapeDtypeStruct((NC, NS, RPT, D_H), jnp.float32),
    mesh=_mesh,
    scratch_types=[
        pltpu.VMEM((CH, K), jnp.int32),
        pltpu.VMEM((K,), jnp.int32),
        pltpu.VMEM((K,), jnp.int32),
        pltpu.VMEM((K, D_H), jnp.float32),
        pltpu.VMEM((K, D_H), jnp.float32),
        pltpu.VMEM_SHARED((N, D_H), jnp.float32),
        pltpu.SemaphoreType.DMA,
        pltpu.SemaphoreType.DMA,
        pltpu.SemaphoreType.DMA,
        pltpu.SemaphoreType.DMA,
        pltpu.SemaphoreType.DMA,
    ],
)
def _sc_scatter(g_hbm, src_hbm, dst_hbm, out_hbm,
                dst_v, si0, si1, rows0, rows1, acc,
                gsem0, gsem1, isem0, isem1, zsem):
    c = lax.axis_index("c")
    s = lax.axis_index("s")
    wid = c * NS + s

    zero = jnp.zeros((16,), jnp.float32)

    def _zfill(i, _):
        for j in range(D_H // 16):
            rows1[i, pl.ds(j * 16, 16)] = zero
        return 0

    lax.fori_loop(0, ZR, _zfill, 0)
    for t in range(RPT // ZR):
        pltpu.async_copy(rows1, acc.at[pl.ds(s * RPT + t * ZR, ZR)], zsem)

    pltpu.sync_copy(dst_hbm.at[wid], dst_v)
    pltpu.sync_copy(src_hbm.at[wid, 0], si0)
    pltpu.sync_copy(src_hbm.at[wid, 1], si1)
    pltpu.async_copy(g_hbm.at[si0], rows0, gsem0)

    for t in range(RPT // ZR):
        pltpu.make_async_copy(
            rows1, acc.at[pl.ds(s * RPT + t * ZR, ZR)], zsem).wait()
    pltpu.async_copy(g_hbm.at[si1], rows1, gsem1)
    plsc.subcore_barrier()

    def _body(jj, _):
        j0 = jj * 2
        j1 = j0 + 1
        pltpu.make_async_copy(g_hbm.at[si0], rows0, gsem0).wait()

        @pl.when(j0 + 2 < CH)
        def _():
            pltpu.async_copy(src_hbm.at[wid, j0 + 2], si0, isem0)

        pltpu.sync_copy(rows0, acc.at[dst_v.at[j0]], add=True)

        @pl.when(j0 + 2 < CH)
        def _():
            pltpu.make_async_copy(src_hbm.at[wid, j0 + 2], si0, isem0).wait()
            pltpu.async_copy(g_hbm.at[si0], rows0, gsem0)

        pltpu.make_async_copy(g_hbm.at[si1], rows1, gsem1).wait()

        @pl.when(j1 + 2 < CH)
        def _():
            pltpu.async_copy(src_hbm.at[wid, j1 + 2], si1, isem1)

        pltpu.sync_copy(rows1, acc.at[dst_v.at[j1]], add=True)

        @pl.when(j1 + 2 < CH)
        def _():
            pltpu.make_async_copy(src_hbm.at[wid, j1 + 2], si1, isem1).wait()
            pltpu.async_copy(g_hbm.at[si1], rows1, gsem1)

        return 0

    lax.fori_loop(0, CH // 2, _body, 0)
    plsc.subcore_barrier()

    pltpu.sync_copy(acc.at[pl.ds(s * RPT, RPT)], out_hbm.at[c, s])


@functools.partial(
    pl.kernel,
    out_type=jax.ShapeDtypeStruct((NC, N), jnp.float32),
    mesh=_mesh,
    scratch_types=[
        pltpu.VMEM((CH, K), jnp.int32),
        pltpu.VMEM((640,), jnp.float32),
        pltpu.VMEM((128,), jnp.float32),
        pltpu.VMEM_SHARED((N,), jnp.float32),
    ],
)
def _sc_degree(dst_hbm, out_hbm, dst_v, zeros_v, ones_v, acc):
    c = lax.axis_index("c")
    s = lax.axis_index("s")
    wid = c * NS + s

    pltpu.sync_copy(dst_hbm.at[wid], dst_v)

    zero = jnp.zeros((16,), jnp.float32)
    one = jnp.ones((16,), jnp.float32)
    for j in range(640 // 16):
        zeros_v[pl.ds(j * 16, 16)] = zero
    for j in range(128 // 16):
        ones_v[pl.ds(j * 16, 16)] = one

    @pl.when(s == 0)
    def _():
        for t in range(15):
            pltpu.sync_copy(zeros_v, acc.at[pl.ds(t * 640, 640)])
        pltpu.sync_copy(zeros_v.at[pl.ds(0, 400)], acc.at[pl.ds(9600, 400)])

    plsc.subcore_barrier()

    def _body(j, _):
        pltpu.sync_copy(ones_v.at[pl.ds(0, K)], acc.at[dst_v.at[j]], add=True)
        return 0

    lax.fori_loop(0, CH, _body, 0)
    plsc.subcore_barrier()

    @pl.when(s == 0)
    def _():
        pltpu.sync_copy(acc, out_hbm.at[c])


R = 2000
GRID = N // R


def _dinv(d0, d1):
    return lax.rsqrt(d0 + d1 + 1.0)


def _enc_body(x_ref, w_ref, b_ref, cw_ref, d0_ref, d1_ref, xo_ref, go_ref):
    x = jnp.dot(x_ref[...], w_ref[...],
                preferred_element_type=jnp.float32) + b_ref[...]
    xo_ref[...] = x
    dinv = _dinv(d0_ref[...], d1_ref[...])
    go_ref[...] = jnp.dot(x, cw_ref[...],
                          preferred_element_type=jnp.float32) * dinv


def _layer_norm(m, lg, lb):
    mu = jnp.mean(m, axis=-1, keepdims=True)
    var = jnp.mean((m - mu) ** 2, axis=-1, keepdims=True)
    return (m - mu) * lax.rsqrt(var + EPS) * lg + lb


def _step_body(x_ref, g_ref, p0_ref, p1_ref, d0_ref, d1_ref,
               cb_ref, lg_ref, lb_ref, cw_ref, xo_ref, go_ref):
    dinv = _dinv(d0_ref[...], d1_ref[...])
    m = x_ref[...] + dinv * (p0_ref[...] + p1_ref[...] + g_ref[...]) + cb_ref[...]
    xn = _layer_norm(m, lg_ref[...], lb_ref[...])
    xo_ref[...] = xn
    go_ref[...] = jnp.dot(xn, cw_ref[...],
                          preferred_element_type=jnp.float32) * dinv


def _final_body(x_ref, g_ref, p0_ref, p1_ref, d0_ref, d1_ref,
                cb_ref, lg_ref, lb_ref, dw_ref, db_ref, o_ref):
    dinv = _dinv(d0_ref[...], d1_ref[...])
    m = x_ref[...] + dinv * (p0_ref[...] + p1_ref[...] + g_ref[...]) + cb_ref[...]
    xn = _layer_norm(m, lg_ref[...], lb_ref[...])
    o_ref[...] = jnp.dot(xn, dw_ref[...],
                         preferred_element_type=jnp.float32) + db_ref[...]


def _row_spec(d):
    return pl.BlockSpec((R, d), lambda i: (i, 0))


def _full_spec(r, d):
    return pl.BlockSpec((r, d), lambda i: (0, 0))


def kernel(X, edge_index, enc_W, enc_b, conv_W, conv_b, ln_g, ln_b, dec_W, dec_b):
    src3 = edge_index[0].reshape(NW, CH, K)
    dst3 = edge_index[1].reshape(NW, CH, K)

    degp = _sc_degree(dst3)
    d0 = degp[0].reshape(N, 1)
    d1 = degp[1].reshape(N, 1)

    enc_b2 = enc_b.reshape(1, D_H)
    conv_b2 = conv_b.reshape(1, D_H)
    ln_g2 = ln_g.reshape(1, D_H)
    ln_b2 = ln_b.reshape(1, D_H)
    dec_b2 = dec_b.reshape(1, D_IN)

    x, g = pl.pallas_call(
        _enc_body,
        grid=(GRID,),
        in_specs=[
            _row_spec(D_IN),
            _full_spec(D_IN, D_H),
            _full_spec(1, D_H),
            _full_spec(D_H, D_H),
            _row_spec(1),
            _row_spec(1),
        ],
        out_specs=[_row_spec(D_H), _row_spec(D_H)],
        out_shape=[
            jax.ShapeDtypeStruct((N, D_H), jnp.float32),
            jax.ShapeDtypeStruct((N, D_H), jnp.float32),
        ],
    )(X, enc_W, enc_b2, conv_W, d0, d1)

    step_call = pl.pallas_call(
        _step_body,
        grid=(GRID,),
        in_specs=[
            _row_spec(D_H), _row_spec(D_H), _row_spec(D_H), _row_spec(D_H),
            _row_spec(1), _row_spec(1),
            _full_spec(1, D_H), _full_spec(1, D_H), _full_spec(1, D_H),
            _full_spec(D_H, D_H),
        ],
        out_specs=[_row_spec(D_H), _row_spec(D_H)],
        out_shape=[
            jax.ShapeDtypeStruct((N, D_H), jnp.float32),
            jax.ShapeDtypeStruct((N, D_H), jnp.float32),
        ],
    )

    final_call = pl.pallas_call(
        _final_body,
        grid=(GRID,),
        in_specs=[
            _row_spec(D_H), _row_spec(D_H), _row_spec(D_H), _row_spec(D_H),
            _row_spec(1), _row_spec(1),
            _full_spec(1, D_H), _full_spec(1, D_H), _full_spec(1, D_H),
            _full_spec(D_H, D_IN), _full_spec(1, D_IN),
        ],
        out_specs=_row_spec(D_IN),
        out_shape=jax.ShapeDtypeStruct((N, D_IN), jnp.float32),
    )

    for step in range(MSG_STEPS):
        part = _sc_scatter(g, src3, dst3)
        p = part.reshape(NC, N, D_H)
        if step < MSG_STEPS - 1:
            x, g = step_call(x, g, p[0], p[1], d0, d1,
                             conv_b2, ln_g2, ln_b2, conv_W)
        else:
            out = final_call(x, g, p[0], p[1], d0, d1,
                             conv_b2, ln_g2, ln_b2, dec_W, dec_b2)
    return out


# speedup vs baseline: 24.5851x; 1.0051x over previous
import functools

import jax
import jax.numpy as jnp
from jax import lax
from jax.experimental import pallas as pl
from jax.experimental.pallas import tpu as pltpu
from jax.experimental.pallas import tpu_sc as plsc

N = 10000
E = 320000
D_IN = 256
D_H = 128
MSG_STEPS = 3
EPS = 1e-5

NC = 2
NS = 16
NW = NC * NS
EPW = E // NW
K = 125
CH = EPW // K
RPT = N // NS
ZR = 125

_mesh = plsc.VectorSubcoreMesh(core_axis_name="c", subcore_axis_name="s")


@functools.partial(
    pl.kernel,
    out_type=jax.ShapeDtypeStruct((NC, NS, RPT, D_H), jnp.float32),
    mesh=_mesh,
    scratch_types=[
        pltpu.VMEM((CH, K), jnp.int32),
        pltpu.VMEM((K,), jnp.int32),
        pltpu.VMEM((K,), jnp.int32),
        pltpu.VMEM((K, D_H), jnp.float32),
        pltpu.VMEM((K, D_H), jnp.float32),
        pltpu.VMEM_SHARED((N, D_H), jnp.float32),
        pltpu.SemaphoreType.DMA,
        pltpu.SemaphoreType.DMA,
        pltpu.SemaphoreType.DMA,
        pltpu.SemaphoreType.DMA,
        pltpu.SemaphoreType.DMA,
    ],
)
def _sc_scatter(g_hbm, src_hbm, dst_hbm, out_hbm,
                dst_v, si0, si1, rows0, rows1, acc,
                gsem0, gsem1, isem0, isem1, zsem):
    c = lax.axis_index("c")
    s = lax.axis_index("s")
    wid = c * NS + s

    zero = jnp.zeros((16,), jnp.float32)

    def _zfill(i, _):
        for j in range(D_H // 16):
            rows1[i, pl.ds(j * 16, 16)] = zero
        return 0

    lax.fori_loop(0, ZR, _zfill, 0)
    for t in range(RPT // ZR):
        pltpu.async_copy(rows1, acc.at[pl.ds(s * RPT + t * ZR, ZR)], zsem)

    pltpu.sync_copy(dst_hbm.at[wid], dst_v)
    pltpu.sync_copy(src_hbm.at[wid, 0], si0)
    pltpu.sync_copy(src_hbm.at[wid, 1], si1)
    pltpu.async_copy(g_hbm.at[si0], rows0, gsem0)

    for t in range(RPT // ZR):
        pltpu.make_async_copy(
            rows1, acc.at[pl.ds(s * RPT + t * ZR, ZR)], zsem).wait()
    pltpu.async_copy(g_hbm.at[si1], rows1, gsem1)
    plsc.subcore_barrier()

    def _body(jj, _):
        j0 = jj * 2
        j1 = j0 + 1
        pltpu.make_async_copy(g_hbm.at[si0], rows0, gsem0).wait()

        @pl.when(j0 + 2 < CH)
        def _():
            pltpu.async_copy(src_hbm.at[wid, j0 + 2], si0, isem0)

        pltpu.sync_copy(rows0, acc.at[dst_v.at[j0]], add=True)

        @pl.when(j0 + 2 < CH)
        def _():
            pltpu.make_async_copy(src_hbm.at[wid, j0 + 2], si0, isem0).wait()
            pltpu.async_copy(g_hbm.at[si0], rows0, gsem0)

        pltpu.make_async_copy(g_hbm.at[si1], rows1, gsem1).wait()

        @pl.when(j1 + 2 < CH)
        def _():
            pltpu.async_copy(src_hbm.at[wid, j1 + 2], si1, isem1)

        pltpu.sync_copy(rows1, acc.at[dst_v.at[j1]], add=True)

        @pl.when(j1 + 2 < CH)
        def _():
            pltpu.make_async_copy(src_hbm.at[wid, j1 + 2], si1, isem1).wait()
            pltpu.async_copy(g_hbm.at[si1], rows1, gsem1)

        return 0

    lax.fori_loop(0, CH // 2, _body, 0)
    plsc.subcore_barrier()

    pltpu.sync_copy(acc.at[pl.ds(s * RPT, RPT)], out_hbm.at[c, s])


@functools.partial(
    pl.kernel,
    out_type=jax.ShapeDtypeStruct((NC, N), jnp.float32),
    mesh=_mesh,
    scratch_types=[
        pltpu.VMEM((CH, K), jnp.int32),
        pltpu.VMEM((640,), jnp.float32),
        pltpu.VMEM((128,), jnp.float32),
        pltpu.VMEM_SHARED((N,), jnp.float32),
    ],
)
def _sc_degree(dst_hbm, out_hbm, dst_v, zeros_v, ones_v, acc):
    c = lax.axis_index("c")
    s = lax.axis_index("s")
    wid = c * NS + s

    pltpu.sync_copy(dst_hbm.at[wid], dst_v)

    zero = jnp.zeros((16,), jnp.float32)
    one = jnp.ones((16,), jnp.float32)
    for j in range(640 // 16):
        zeros_v[pl.ds(j * 16, 16)] = zero
    for j in range(128 // 16):
        ones_v[pl.ds(j * 16, 16)] = one

    @pl.when(s == 0)
    def _():
        for t in range(15):
            pltpu.sync_copy(zeros_v, acc.at[pl.ds(t * 640, 640)])
        pltpu.sync_copy(zeros_v.at[pl.ds(0, 400)], acc.at[pl.ds(9600, 400)])

    plsc.subcore_barrier()

    def _body(j, _):
        pltpu.sync_copy(ones_v.at[pl.ds(0, K)], acc.at[dst_v.at[j]], add=True)
        return 0

    lax.fori_loop(0, CH, _body, 0)
    plsc.subcore_barrier()

    @pl.when(s == 0)
    def _():
        pltpu.sync_copy(acc, out_hbm.at[c])


R = 2000
GRID = N // R


def _dinv(d0, d1):
    return lax.rsqrt(d0 + d1 + 1.0)


def _enc_body(x_ref, w_ref, b_ref, cw_ref, xo_ref, ho_ref):
    x = jnp.dot(x_ref[...], w_ref[...],
                preferred_element_type=jnp.float32) + b_ref[...]
    xo_ref[...] = x
    ho_ref[...] = jnp.dot(x, cw_ref[...], preferred_element_type=jnp.float32)


def _mul_body(h_ref, d0_ref, d1_ref, go_ref):
    go_ref[...] = h_ref[...] * _dinv(d0_ref[...], d1_ref[...])


def _layer_norm(m, lg, lb):
    mu = jnp.mean(m, axis=-1, keepdims=True)
    var = jnp.mean((m - mu) ** 2, axis=-1, keepdims=True)
    return (m - mu) * lax.rsqrt(var + EPS) * lg + lb


def _step_body(x_ref, g_ref, p0_ref, p1_ref, d0_ref, d1_ref,
               cb_ref, lg_ref, lb_ref, cw_ref, xo_ref, go_ref):
    dinv = _dinv(d0_ref[...], d1_ref[...])
    m = x_ref[...] + dinv * (p0_ref[...] + p1_ref[...] + g_ref[...]) + cb_ref[...]
    xn = _layer_norm(m, lg_ref[...], lb_ref[...])
    xo_ref[...] = xn
    go_ref[...] = jnp.dot(xn, cw_ref[...],
                          preferred_element_type=jnp.float32) * dinv


def _final_body(x_ref, g_ref, p0_ref, p1_ref, d0_ref, d1_ref,
                cb_ref, lg_ref, lb_ref, dw_ref, db_ref, o_ref):
    dinv = _dinv(d0_ref[...], d1_ref[...])
    m = x_ref[...] + dinv * (p0_ref[...] + p1_ref[...] + g_ref[...]) + cb_ref[...]
    xn = _layer_norm(m, lg_ref[...], lb_ref[...])
    o_ref[...] = jnp.dot(xn, dw_ref[...],
                         preferred_element_type=jnp.float32) + db_ref[...]


def _row_spec(d):
    return pl.BlockSpec((R, d), lambda i: (i, 0))


def _full_spec(r, d):
    return pl.BlockSpec((r, d), lambda i: (0, 0))


def kernel(X, edge_index, enc_W, enc_b, conv_W, conv_b, ln_g, ln_b, dec_W, dec_b):
    src3 = edge_index[0].reshape(NW, CH, K)
    dst3 = edge_index[1].reshape(NW, CH, K)

    degp = _sc_degree(dst3)
    d0 = degp[0].reshape(N, 1)
    d1 = degp[1].reshape(N, 1)

    enc_b2 = enc_b.reshape(1, D_H)
    conv_b2 = conv_b.reshape(1, D_H)
    ln_g2 = ln_g.reshape(1, D_H)
    ln_b2 = ln_b.reshape(1, D_H)
    dec_b2 = dec_b.reshape(1, D_IN)

    x, h = pl.pallas_call(
        _enc_body,
        grid=(GRID,),
        in_specs=[
            _row_spec(D_IN),
            _full_spec(D_IN, D_H),
            _full_spec(1, D_H),
            _full_spec(D_H, D_H),
        ],
        out_specs=[_row_spec(D_H), _row_spec(D_H)],
        out_shape=[
            jax.ShapeDtypeStruct((N, D_H), jnp.float32),
            jax.ShapeDtypeStruct((N, D_H), jnp.float32),
        ],
    )(X, enc_W, enc_b2, conv_W)

    g = pl.pallas_call(
        _mul_body,
        grid=(GRID,),
        in_specs=[_row_spec(D_H), _row_spec(1), _row_spec(1)],
        out_specs=_row_spec(D_H),
        out_shape=jax.ShapeDtypeStruct((N, D_H), jnp.float32),
    )(h, d0, d1)

    step_call = pl.pallas_call(
        _step_body,
        grid=(GRID,),
        in_specs=[
            _row_spec(D_H), _row_spec(D_H), _row_spec(D_H), _row_spec(D_H),
            _row_spec(1), _row_spec(1),
            _full_spec(1, D_H), _full_spec(1, D_H), _full_spec(1, D_H),
            _full_spec(D_H, D_H),
        ],
        out_specs=[_row_spec(D_H), _row_spec(D_H)],
        out_shape=[
            jax.ShapeDtypeStruct((N, D_H), jnp.float32),
            jax.ShapeDtypeStruct((N, D_H), jnp.float32),
        ],
    )

    final_call = pl.pallas_call(
        _final_body,
        grid=(GRID,),
        in_specs=[
            _row_spec(D_H), _row_spec(D_H), _row_spec(D_H), _row_spec(D_H),
            _row_spec(1), _row_spec(1),
            _full_spec(1, D_H), _full_spec(1, D_H), _full_spec(1, D_H),
            _full_spec(D_H, D_IN), _full_spec(1, D_IN),
        ],
        out_specs=_row_spec(D_IN),
        out_shape=jax.ShapeDtypeStruct((N, D_IN), jnp.float32),
    )

    for step in range(MSG_STEPS):
        part = _sc_scatter(g, src3, dst3)
        p = part.reshape(NC, N, D_H)
        if step < MSG_STEPS - 1:
            x, g = step_call(x, g, p[0], p[1], d0, d1,
                             conv_b2, ln_g2, ln_b2, conv_W)
        else:
            out = final_call(x, g, p[0], p[1], d0, d1,
                             conv_b2, ln_g2, ln_b2, dec_W, dec_b2)
    return out


# speedup vs baseline: 24.7342x; 1.0061x over previous
import functools

import jax
import jax.numpy as jnp
from jax import lax
from jax.experimental import pallas as pl
from jax.experimental.pallas import tpu as pltpu
from jax.experimental.pallas import tpu_sc as plsc

N = 10000
E = 320000
D_IN = 256
D_H = 128
MSG_STEPS = 3
EPS = 1e-5

NC = 2
NS = 16
NW = NC * NS
EPW = E // NW
K = 125
CH = EPW // K
RPT = N // NS
ZR = 125

_mesh = plsc.VectorSubcoreMesh(core_axis_name="c", subcore_axis_name="s")


@functools.partial(
    pl.kernel,
    out_type=jax.ShapeDtypeStruct((NC, NS, RPT, D_H), jnp.float32),
    mesh=_mesh,
    scratch_types=[
        pltpu.VMEM((CH, K), jnp.int32),
        [pltpu.VMEM((K,), jnp.int32) for _ in range(4)],
        pltpu.VMEM((K, D_H), jnp.float32),
        pltpu.VMEM((K, D_H), jnp.float32),
        pltpu.VMEM_SHARED((N, D_H), jnp.float32),
        pltpu.SemaphoreType.DMA,
        pltpu.SemaphoreType.DMA,
        [pltpu.SemaphoreType.DMA for _ in range(4)],
        pltpu.SemaphoreType.DMA,
    ],
)
def _sc_scatter(g_hbm, src_hbm, dst_hbm, out_hbm,
                dst_v, si, rows0, rows1, acc,
                gsem0, gsem1, isem, zsem):
    c = lax.axis_index("c")
    s = lax.axis_index("s")
    wid = c * NS + s

    zero = jnp.zeros((16,), jnp.float32)

    def _zfill(i, _):
        for j in range(D_H // 16):
            rows1[i, pl.ds(j * 16, 16)] = zero
        return 0

    lax.fori_loop(0, ZR, _zfill, 0)
    for t in range(RPT // ZR):
        pltpu.async_copy(rows1, acc.at[pl.ds(s * RPT + t * ZR, ZR)], zsem)

    pltpu.sync_copy(dst_hbm.at[wid], dst_v)
    for q in range(4):
        pltpu.async_copy(src_hbm.at[wid, q], si[q], isem[q])
    pltpu.make_async_copy(src_hbm.at[wid, 0], si[0], isem[0]).wait()
    pltpu.async_copy(g_hbm.at[si[0]], rows0, gsem0)
    pltpu.make_async_copy(src_hbm.at[wid, 1], si[1], isem[1]).wait()

    for t in range(RPT // ZR):
        pltpu.make_async_copy(
            rows1, acc.at[pl.ds(s * RPT + t * ZR, ZR)], zsem).wait()
    pltpu.async_copy(g_hbm.at[si[1]], rows1, gsem1)
    plsc.subcore_barrier()

    def _body(jj, _):
        j0 = jj * 4
        rr = (rows0, rows1)
        gg = (gsem0, gsem1)
        for u in range(4):
            ch = j0 + u
            rbuf = rr[u % 2]
            gsem = gg[u % 2]
            pltpu.make_async_copy(g_hbm.at[si[u]], rbuf, gsem).wait()
            pltpu.sync_copy(rbuf, acc.at[dst_v.at[ch]], add=True)

            @pl.when(ch + 2 < CH)
            def _():
                pltpu.make_async_copy(
                    src_hbm.at[wid, ch + 2], si[(u + 2) % 4], isem[(u + 2) % 4]
                ).wait()
                pltpu.async_copy(g_hbm.at[si[(u + 2) % 4]], rbuf, gsem)

            @pl.when(ch + 4 < CH)
            def _():
                pltpu.async_copy(
                    src_hbm.at[wid, ch + 4], si[u], isem[u])

        return 0

    lax.fori_loop(0, CH // 4, _body, 0)
    plsc.subcore_barrier()

    pltpu.sync_copy(acc.at[pl.ds(s * RPT, RPT)], out_hbm.at[c, s])


@functools.partial(
    pl.kernel,
    out_type=jax.ShapeDtypeStruct((NC, N), jnp.float32),
    mesh=_mesh,
    scratch_types=[
        pltpu.VMEM((CH, K), jnp.int32),
        pltpu.VMEM((640,), jnp.float32),
        pltpu.VMEM((128,), jnp.float32),
        pltpu.VMEM_SHARED((N,), jnp.float32),
    ],
)
def _sc_degree(dst_hbm, out_hbm, dst_v, zeros_v, ones_v, acc):
    c = lax.axis_index("c")
    s = lax.axis_index("s")
    wid = c * NS + s

    pltpu.sync_copy(dst_hbm.at[wid], dst_v)

    zero = jnp.zeros((16,), jnp.float32)
    one = jnp.ones((16,), jnp.float32)
    for j in range(640 // 16):
        zeros_v[pl.ds(j * 16, 16)] = zero
    for j in range(128 // 16):
        ones_v[pl.ds(j * 16, 16)] = one

    @pl.when(s == 0)
    def _():
        for t in range(15):
            pltpu.sync_copy(zeros_v, acc.at[pl.ds(t * 640, 640)])
        pltpu.sync_copy(zeros_v.at[pl.ds(0, 400)], acc.at[pl.ds(9600, 400)])

    plsc.subcore_barrier()

    def _body(j, _):
        pltpu.sync_copy(ones_v.at[pl.ds(0, K)], acc.at[dst_v.at[j]], add=True)
        return 0

    lax.fori_loop(0, CH, _body, 0)
    plsc.subcore_barrier()

    @pl.when(s == 0)
    def _():
        pltpu.sync_copy(acc, out_hbm.at[c])


R = 2000
GRID = N // R


def _dinv(d0, d1):
    return lax.rsqrt(d0 + d1 + 1.0)


def _enc_body(x_ref, w_ref, b_ref, cw_ref, xo_ref, ho_ref):
    x = jnp.dot(x_ref[...], w_ref[...],
                preferred_element_type=jnp.float32) + b_ref[...]
    xo_ref[...] = x
    ho_ref[...] = jnp.dot(x, cw_ref[...], preferred_element_type=jnp.float32)


def _mul_body(h_ref, d0_ref, d1_ref, go_ref):
    go_ref[...] = h_ref[...] * _dinv(d0_ref[...], d1_ref[...])


def _layer_norm(m, lg, lb):
    mu = jnp.mean(m, axis=-1, keepdims=True)
    var = jnp.mean((m - mu) ** 2, axis=-1, keepdims=True)
    return (m - mu) * lax.rsqrt(var + EPS) * lg + lb


def _step_body(x_ref, g_ref, p0_ref, p1_ref, d0_ref, d1_ref,
               cb_ref, lg_ref, lb_ref, cw_ref, xo_ref, go_ref):
    dinv = _dinv(d0_ref[...], d1_ref[...])
    m = x_ref[...] + dinv * (p0_ref[...] + p1_ref[...] + g_ref[...]) + cb_ref[...]
    xn = _layer_norm(m, lg_ref[...], lb_ref[...])
    xo_ref[...] = xn
    go_ref[...] = jnp.dot(xn, cw_ref[...],
                          preferred_element_type=jnp.float32) * dinv


def _final_body(x_ref, g_ref, p0_ref, p1_ref, d0_ref, d1_ref,
                cb_ref, lg_ref, lb_ref, dw_ref, db_ref, o_ref):
    dinv = _dinv(d0_ref[...], d1_ref[...])
    m = x_ref[...] + dinv * (p0_ref[...] + p1_ref[...] + g_ref[...]) + cb_ref[...]
    xn = _layer_norm(m, lg_ref[...], lb_ref[...])
    o_ref[...] = jnp.dot(xn, dw_ref[...],
                         preferred_element_type=jnp.float32) + db_ref[...]


def _row_spec(d):
    return pl.BlockSpec((R, d), lambda i: (i, 0))


def _full_spec(r, d):
    return pl.BlockSpec((r, d), lambda i: (0, 0))


def kernel(X, edge_index, enc_W, enc_b, conv_W, conv_b, ln_g, ln_b, dec_W, dec_b):
    src3 = edge_index[0].reshape(NW, CH, K)
    dst3 = edge_index[1].reshape(NW, CH, K)

    degp = _sc_degree(dst3)
    d0 = degp[0].reshape(N, 1)
    d1 = degp[1].reshape(N, 1)

    enc_b2 = enc_b.reshape(1, D_H)
    conv_b2 = conv_b.reshape(1, D_H)
    ln_g2 = ln_g.reshape(1, D_H)
    ln_b2 = ln_b.reshape(1, D_H)
    dec_b2 = dec_b.reshape(1, D_IN)

    x, h = pl.pallas_call(
        _enc_body,
        grid=(GRID,),
        in_specs=[
            _row_spec(D_IN),
            _full_spec(D_IN, D_H),
            _full_spec(1, D_H),
            _full_spec(D_H, D_H),
        ],
        out_specs=[_row_spec(D_H), _row_spec(D_H)],
        out_shape=[
            jax.ShapeDtypeStruct((N, D_H), jnp.float32),
            jax.ShapeDtypeStruct((N, D_H), jnp.float32),
        ],
    )(X, enc_W, enc_b2, conv_W)

    g = pl.pallas_call(
        _mul_body,
        grid=(GRID,),
        in_specs=[_row_spec(D_H), _row_spec(1), _row_spec(1)],
        out_specs=_row_spec(D_H),
        out_shape=jax.ShapeDtypeStruct((N, D_H), jnp.float32),
    )(h, d0, d1)

    step_call = pl.pallas_call(
        _step_body,
        grid=(GRID,),
        in_specs=[
            _row_spec(D_H), _row_spec(D_H), _row_spec(D_H), _row_spec(D_H),
            _row_spec(1), _row_spec(1),
            _full_spec(1, D_H), _full_spec(1, D_H), _full_spec(1, D_H),
            _full_spec(D_H, D_H),
        ],
        out_specs=[_row_spec(D_H), _row_spec(D_H)],
        out_shape=[
            jax.ShapeDtypeStruct((N, D_H), jnp.float32),
            jax.ShapeDtypeStruct((N, D_H), jnp.float32),
        ],
    )

    final_call = pl.pallas_call(
        _final_body,
        grid=(GRID,),
        in_specs=[
            _row_spec(D_H), _row_spec(D_H), _row_spec(D_H), _row_spec(D_H),
            _row_spec(1), _row_spec(1),
            _full_spec(1, D_H), _full_spec(1, D_H), _full_spec(1, D_H),
            _full_spec(D_H, D_IN), _full_spec(1, D_IN),
        ],
        out_specs=_row_spec(D_IN),
        out_shape=jax.ShapeDtypeStruct((N, D_IN), jnp.float32),
    )

    for step in range(MSG_STEPS):
        part = _sc_scatter(g, src3, dst3)
        p = part.reshape(NC, N, D_H)
        if step < MSG_STEPS - 1:
            x, g = step_call(x, g, p[0], p[1], d0, d1,
                             conv_b2, ln_g2, ln_b2, conv_W)
        else:
            out = final_call(x, g, p[0], p[1], d0, d1,
                             conv_b2, ln_g2, ln_b2, dec_W, dec_b2)
    return out
